# baseline (speedup 1.0000x reference)
import functools

import jax
import jax.numpy as jnp
from jax import lax
from jax.experimental import pallas as pl
from jax.experimental.pallas import tpu as pltpu
from jax.experimental.pallas import tpu_sc as plsc

_EPS = 1e-5
_B = 16384
_F = 26
_V = 100000
_D = 50
_NC = 13
_L1 = 512
_L2 = 256

_DP = 128
_H = 64
_NPAIR = _F * _V // 2

_ROWS = _B * _F
_NUM_WORKERS = 32
_CHUNK = 256
_PGROUPS = (7, 6)


def _sc_gather(nf, tab128, flat_idx):
    rows = nf * _B
    rows_per_w = rows // _NUM_WORKERS
    nchunks = rows_per_w // _CHUNK
    mesh = plsc.VectorSubcoreMesh(core_axis_name="c", subcore_axis_name="s")

    @functools.partial(
        pl.kernel,
        out_type=jax.ShapeDtypeStruct((rows, _DP), jnp.float32),
        mesh=mesh,
        scratch_types=[
            pltpu.VMEM((rows_per_w,), jnp.int32),
            pltpu.VMEM((_CHUNK, _DP), jnp.float32),
            pltpu.VMEM((_CHUNK, _DP), jnp.float32),
            pltpu.SemaphoreType.DMA,
            pltpu.SemaphoreType.DMA,
        ],
        compiler_params=pltpu.CompilerParams(use_tc_tiling_on_sc=True),
    )
    def gather_kernel(tab_hbm, idx_hbm, out_hbm, idx_v, buf0, buf1, sem0,
                      sem1):
        wid = lax.axis_index("s") * 2 + lax.axis_index("c")
        base = wid * rows_per_w
        pltpu.sync_copy(idx_hbm.at[pl.ds(base, rows_per_w)], idx_v)

        def start(i, buf, sem):
            pltpu.async_copy(
                tab_hbm.at[idx_v.at[pl.ds(i * _CHUNK, _CHUNK)]], buf, sem)

        def finish(i, buf, sem):
            pltpu.make_async_copy(
                tab_hbm.at[idx_v.at[pl.ds(i * _CHUNK, _CHUNK)]], buf, sem
            ).wait()
            pltpu.sync_copy(buf, out_hbm.at[pl.ds(base + i * _CHUNK, _CHUNK)])

        start(0, buf0, sem0)

        @pl.loop(0, nchunks, step=2)
        def _(i):
            start(i + 1, buf1, sem1)
            finish(i, buf0, sem0)

            @pl.when(i + 2 < nchunks)
            def _():
                start(i + 2, buf0, sem0)

            finish(i + 1, buf1, sem1)

    return gather_kernel(tab128, flat_idx)


def _mlp_body(x0_ref, x1_ref, xc_ref, gc_ref, bc_ref, w1a_ref, w1b_ref,
              b1_ref, g1_ref, bt1_ref, w2_ref, b2_ref, g2_ref, bt2_ref,
              wo_ref, bo_ref, o_ref):
    inv = (1.0 / jnp.sqrt(1.0 + _EPS)).astype(jnp.float32)
    xc = xc_ref[...] * (gc_ref[...] * inv) + bc_ref[...]
    xw = jnp.concatenate(
        [x0_ref[f] for f in range(2 * _PGROUPS[0])]
        + [x1_ref[f] for f in range(2 * _PGROUPS[1])], axis=-1)
    h = jnp.dot(xw, w1a_ref[...], preferred_element_type=jnp.float32)
    h = h + jnp.dot(xc, w1b_ref[...], preferred_element_type=jnp.float32)
    h = jnp.maximum(h + b1_ref[...], 0.0)
    h = h * (g1_ref[...] * inv) + bt1_ref[...]
    h = jnp.maximum(
        jnp.dot(h, w2_ref[...], preferred_element_type=jnp.float32)
        + b2_ref[...], 0.0)
    h = h * (g2_ref[...] * inv) + bt2_ref[...]
    o_ref[...] = (
        jnp.dot(h, wo_ref[...], preferred_element_type=jnp.float32)
        + bo_ref[...])


def _tc_mlp(xs, x_cont, g_cont, b_cont, W1, b1, g1, beta1, W2, b2, g2,
            beta2, Wo, bo):
    bt = 1024
    grid = (_B // bt,)
    row = lambda v: v.reshape(1, -1)
    w1h = jnp.pad(W1[:_F * _D].reshape(_F, _D, _L1),
                  ((0, 0), (0, _H - _D), (0, 0)))
    zero = jnp.zeros_like(w1h)
    w1a = jnp.where(
        (jnp.arange(_F) % 2 == 0)[:, None, None],
        jnp.concatenate([w1h, zero], axis=1),
        jnp.concatenate([zero, w1h], axis=1)).reshape(_F * _DP, _L1)
    args = tuple(xs) + (
        x_cont, row(g_cont), row(b_cont),
        w1a, W1[_F * _D:], row(b1), row(g1), row(beta1),
        W2, row(b2), row(g2), row(beta2), Wo, row(bo),
    )
    full = lambda a: pl.BlockSpec(a.shape, lambda i: (0,) * a.ndim)
    in_specs = [
        pl.BlockSpec((2 * np, bt, _DP), lambda i: (0, i, 0))
        for np in _PGROUPS
    ] + [
        pl.BlockSpec((bt, _NC), lambda i: (i, 0)),
    ] + [full(a) for a in args[3:]]
    return pl.pallas_call(
        _mlp_body,
        grid=grid,
        in_specs=in_specs,
        out_specs=pl.BlockSpec((bt, 1), lambda i: (i, 0)),
        out_shape=jax.ShapeDtypeStruct((_B, 1), jnp.float32),
        compiler_params=pltpu.CompilerParams(
            dimension_semantics=("arbitrary",)),
    )(*args)


def kernel(x_cat, x_cont, tables, g_cont, b_cont, W1, b1, g1, beta1, W2, b2,
           g2, beta2, Wo, bo):
    view = jnp.transpose(tables, (0, 2, 1)).reshape(_F // 2, 2, _D, _V)
    eye = jnp.eye(_D, _DP, dtype=jnp.float32)
    rhs = jnp.stack([eye, jnp.roll(eye, _H, axis=1)], axis=0)
    rhs = lax.optimization_barrier(rhs)
    xcat_t = x_cat.astype(jnp.int32).T
    xs = []
    p0 = 0
    for np_ in _PGROUPS:
        nf = 2 * np_
        tab = jnp.einsum("fpdv,pdc->fvc", view[p0:p0 + np_], rhs,
                         precision=lax.Precision.DEFAULT)
        tab = tab.reshape(np_ * _V, _DP)
        offs = ((jnp.arange(nf, dtype=jnp.int32) // 2) * _V)[:, None]
        idx = (xcat_t[2 * p0:2 * p0 + nf] + offs).reshape(nf * _B)
        emb = _sc_gather(nf, tab, idx)
        xs.append(emb.reshape(nf, _B, _DP))
        p0 += np_
    return _tc_mlp(xs, x_cont, g_cont, b_cont, W1, b1, g1, beta1, W2,
                   b2, g2, beta2, Wo, bo)


# speedup vs baseline: 1.1432x; 1.1432x over previous
import functools

import jax
import jax.numpy as jnp
from jax import lax
from jax.experimental import pallas as pl
from jax.experimental.pallas import tpu as pltpu
from jax.experimental.pallas import tpu_sc as plsc

_EPS = 1e-5
_B = 16384
_F = 26
_V = 100000
_D = 50
_NC = 13
_L1 = 512
_L2 = 256

_DP = 128
_H = 64
_NPAIR = _F * _V // 2

_ROWS = _B * _F
_NUM_WORKERS = 32
_ROWS_PER_W = _ROWS // _NUM_WORKERS
_CHUNK = 256
_NCHUNKS = _ROWS_PER_W // _CHUNK


def _sc_gather(tab128, flat_idx):
    mesh = plsc.VectorSubcoreMesh(core_axis_name="c", subcore_axis_name="s")

    @functools.partial(
        pl.kernel,
        out_type=jax.ShapeDtypeStruct((_ROWS, _DP), jnp.float32),
        mesh=mesh,
        scratch_types=[
            pltpu.VMEM((_ROWS_PER_W,), jnp.int32),
            pltpu.VMEM((_CHUNK, _DP), jnp.float32),
            pltpu.VMEM((_CHUNK, _DP), jnp.float32),
            pltpu.SemaphoreType.DMA,
            pltpu.SemaphoreType.DMA,
        ],
        compiler_params=pltpu.CompilerParams(use_tc_tiling_on_sc=True),
    )
    def gather_kernel(tab_hbm, idx_hbm, out_hbm, idx_v, buf0, buf1, sem0,
                      sem1):
        wid = lax.axis_index("s") * 2 + lax.axis_index("c")
        base = wid * _ROWS_PER_W
        pltpu.sync_copy(idx_hbm.at[pl.ds(base, _ROWS_PER_W)], idx_v)

        def start(i, buf, sem):
            pltpu.async_copy(
                tab_hbm.at[idx_v.at[pl.ds(i * _CHUNK, _CHUNK)]], buf, sem)

        def finish(i, buf, sem):
            pltpu.make_async_copy(
                tab_hbm.at[idx_v.at[pl.ds(i * _CHUNK, _CHUNK)]], buf, sem
            ).wait()
            pltpu.sync_copy(buf, out_hbm.at[pl.ds(base + i * _CHUNK, _CHUNK)])

        start(0, buf0, sem0)

        @pl.loop(0, _NCHUNKS, step=2)
        def _(i):
            start(i + 1, buf1, sem1)
            finish(i, buf0, sem0)

            @pl.when(i + 2 < _NCHUNKS)
            def _():
                start(i + 2, buf0, sem0)

            finish(i + 1, buf1, sem1)

    return gather_kernel(tab128, flat_idx)


def _mlp_body(x_ref, xc_ref, gc_ref, bc_ref, w1a_ref, w1b_ref,
              b1_ref, g1_ref, bt1_ref, w2_ref, b2_ref, g2_ref, bt2_ref,
              wo_ref, bo_ref, o_ref):
    inv = (1.0 / jnp.sqrt(1.0 + _EPS)).astype(jnp.float32)
    xc = xc_ref[...] * (gc_ref[...] * inv) + bc_ref[...]
    xw = jnp.concatenate([x_ref[f] for f in range(_F)], axis=-1)
    h = jnp.dot(xw, w1a_ref[...], preferred_element_type=jnp.float32)
    h = h + jnp.dot(xc, w1b_ref[...], preferred_element_type=jnp.float32)
    h = jnp.maximum(h + b1_ref[...], 0.0)
    h = h * (g1_ref[...] * inv) + bt1_ref[...]
    h = jnp.maximum(
        jnp.dot(h, w2_ref[...], preferred_element_type=jnp.float32)
        + b2_ref[...], 0.0)
    h = h * (g2_ref[...] * inv) + bt2_ref[...]
    o_ref[...] = (
        jnp.dot(h, wo_ref[...], preferred_element_type=jnp.float32)
        + bo_ref[...])


def _tc_mlp(x, x_cont, g_cont, b_cont, W1, b1, g1, beta1, W2, b2, g2,
            beta2, Wo, bo):
    bt = 1024
    grid = (_B // bt,)
    row = lambda v: v.reshape(1, -1)
    w1h = jnp.pad(W1[:_F * _D].reshape(_F, _D, _L1),
                  ((0, 0), (0, _H - _D), (0, 0)))
    zero = jnp.zeros_like(w1h)
    w1a = jnp.where(
        (jnp.arange(_F) % 2 == 0)[:, None, None],
        jnp.concatenate([w1h, zero], axis=1),
        jnp.concatenate([zero, w1h], axis=1)).reshape(_F * _DP, _L1)
    args = (
        x, x_cont, row(g_cont), row(b_cont),
        w1a, W1[_F * _D:], row(b1), row(g1), row(beta1),
        W2, row(b2), row(g2), row(beta2), Wo, row(bo),
    )
    full = lambda a: pl.BlockSpec(a.shape, lambda i: (0,) * a.ndim)
    in_specs = [
        pl.BlockSpec((_F, bt, _DP), lambda i: (0, i, 0)),
        pl.BlockSpec((bt, _NC), lambda i: (i, 0)),
    ] + [full(a) for a in args[2:]]
    return pl.pallas_call(
        _mlp_body,
        grid=grid,
        in_specs=in_specs,
        out_specs=pl.BlockSpec((bt, 1), lambda i: (i, 0)),
        out_shape=jax.ShapeDtypeStruct((_B, 1), jnp.float32),
        compiler_params=pltpu.CompilerParams(
            dimension_semantics=("arbitrary",)),
    )(*args)


def kernel(x_cat, x_cont, tables, g_cont, b_cont, W1, b1, g1, beta1, W2, b2,
           g2, beta2, Wo, bo):
    view = jnp.transpose(tables, (0, 2, 1)).reshape(_F // 2, 2, _D, _V)
    eye = jnp.eye(_D, _DP, dtype=jnp.float32)
    rhs = jnp.stack([eye, jnp.roll(eye, _H, axis=1)], axis=0)
    rhs = lax.optimization_barrier(rhs)
    tab128 = jnp.einsum("fpdv,pdc->fvc", view, rhs,
                        precision=lax.Precision.DEFAULT).reshape(_NPAIR, _DP)
    xc32 = x_cat.astype(jnp.int32)
    offs = ((jnp.arange(_F, dtype=jnp.int32) // 2) * _V)[:, None]
    flat_idx = (xc32.T + offs).reshape(_ROWS)
    emb = _sc_gather(tab128, flat_idx)
    x = emb.reshape(_F, _B, _DP)
    return _tc_mlp(x, x_cont, g_cont, b_cont, W1, b1, g1, beta1, W2,
                   b2, g2, beta2, Wo, bo)


# speedup vs baseline: 1.1439x; 1.0006x over previous
import functools

import jax
import jax.numpy as jnp
from jax import lax
from jax.experimental import pallas as pl
from jax.experimental.pallas import tpu as pltpu
from jax.experimental.pallas import tpu_sc as plsc

_EPS = 1e-5
_B = 16384
_F = 26
_V = 100000
_D = 50
_NC = 13
_L1 = 512
_L2 = 256

_DP = 128
_H = 64
_NPAIR = _F * _V // 2

_ROWS = _B * _F
_NUM_WORKERS = 32
_ROWS_PER_W = _ROWS // _NUM_WORKERS
_CHUNK = 416
_NCHUNKS = _ROWS_PER_W // _CHUNK


def _sc_gather(tab128, flat_idx):
    mesh = plsc.VectorSubcoreMesh(core_axis_name="c", subcore_axis_name="s")

    @functools.partial(
        pl.kernel,
        out_type=jax.ShapeDtypeStruct((_ROWS, _DP), jnp.float32),
        mesh=mesh,
        scratch_types=[
            pltpu.VMEM((_ROWS_PER_W,), jnp.int32),
            pltpu.VMEM((_CHUNK, _DP), jnp.float32),
            pltpu.VMEM((_CHUNK, _DP), jnp.float32),
            pltpu.SemaphoreType.DMA,
            pltpu.SemaphoreType.DMA,
        ],
        compiler_params=pltpu.CompilerParams(use_tc_tiling_on_sc=True),
    )
    def gather_kernel(tab_hbm, idx_hbm, out_hbm, idx_v, buf0, buf1, sem0,
                      sem1):
        wid = lax.axis_index("s") * 2 + lax.axis_index("c")
        base = wid * _ROWS_PER_W
        pltpu.sync_copy(idx_hbm.at[pl.ds(base, _ROWS_PER_W)], idx_v)

        def start(i, buf, sem):
            pltpu.async_copy(
                tab_hbm.at[idx_v.at[pl.ds(i * _CHUNK, _CHUNK)]], buf, sem)

        def finish(i, buf, sem):
            pltpu.make_async_copy(
                tab_hbm.at[idx_v.at[pl.ds(i * _CHUNK, _CHUNK)]], buf, sem
            ).wait()
            pltpu.sync_copy(buf, out_hbm.at[pl.ds(base + i * _CHUNK, _CHUNK)])

        start(0, buf0, sem0)

        @pl.loop(0, _NCHUNKS, step=2)
        def _(i):
            start(i + 1, buf1, sem1)
            finish(i, buf0, sem0)

            @pl.when(i + 2 < _NCHUNKS)
            def _():
                start(i + 2, buf0, sem0)

            finish(i + 1, buf1, sem1)

    return gather_kernel(tab128, flat_idx)


def _mlp_body(x_ref, xc_ref, gc_ref, bc_ref, w1a_ref, w1b_ref,
              b1_ref, g1_ref, bt1_ref, w2_ref, b2_ref, g2_ref, bt2_ref,
              wo_ref, bo_ref, o_ref):
    inv = (1.0 / jnp.sqrt(1.0 + _EPS)).astype(jnp.float32)
    xc = xc_ref[...] * (gc_ref[...] * inv) + bc_ref[...]
    xw = jnp.concatenate([x_ref[f] for f in range(_F)], axis=-1)
    h = jnp.dot(xw, w1a_ref[...], preferred_element_type=jnp.float32)
    h = h + jnp.dot(xc, w1b_ref[...], preferred_element_type=jnp.float32)
    h = jnp.maximum(h + b1_ref[...], 0.0)
    h = h * (g1_ref[...] * inv) + bt1_ref[...]
    h = jnp.maximum(
        jnp.dot(h, w2_ref[...], preferred_element_type=jnp.float32)
        + b2_ref[...], 0.0)
    h = h * (g2_ref[...] * inv) + bt2_ref[...]
    o_ref[...] = (
        jnp.dot(h, wo_ref[...], preferred_element_type=jnp.float32)
        + bo_ref[...])


def _tc_mlp(x, x_cont, g_cont, b_cont, W1, b1, g1, beta1, W2, b2, g2,
            beta2, Wo, bo):
    bt = 1024
    grid = (_B // bt,)
    row = lambda v: v.reshape(1, -1)
    w1h = jnp.pad(W1[:_F * _D].reshape(_F, _D, _L1),
                  ((0, 0), (0, _H - _D), (0, 0)))
    zero = jnp.zeros_like(w1h)
    w1a = jnp.where(
        (jnp.arange(_F) % 2 == 0)[:, None, None],
        jnp.concatenate([w1h, zero], axis=1),
        jnp.concatenate([zero, w1h], axis=1)).reshape(_F * _DP, _L1)
    args = (
        x, x_cont, row(g_cont), row(b_cont),
        w1a, W1[_F * _D:], row(b1), row(g1), row(beta1),
        W2, row(b2), row(g2), row(beta2), Wo, row(bo),
    )
    full = lambda a: pl.BlockSpec(a.shape, lambda i: (0,) * a.ndim)
    in_specs = [
        pl.BlockSpec((_F, bt, _DP), lambda i: (0, i, 0)),
        pl.BlockSpec((bt, _NC), lambda i: (i, 0)),
    ] + [full(a) for a in args[2:]]
    return pl.pallas_call(
        _mlp_body,
        grid=grid,
        in_specs=in_specs,
        out_specs=pl.BlockSpec((bt, 1), lambda i: (i, 0)),
        out_shape=jax.ShapeDtypeStruct((_B, 1), jnp.float32),
        compiler_params=pltpu.CompilerParams(
            dimension_semantics=("arbitrary",)),
    )(*args)


def kernel(x_cat, x_cont, tables, g_cont, b_cont, W1, b1, g1, beta1, W2, b2,
           g2, beta2, Wo, bo):
    view = jnp.transpose(tables, (0, 2, 1)).reshape(_F // 2, 2, _D, _V)
    eye = jnp.eye(_D, _DP, dtype=jnp.float32)
    rhs = jnp.stack([eye, jnp.roll(eye, _H, axis=1)], axis=0)
    rhs = lax.optimization_barrier(rhs)
    tab128 = jnp.einsum("fpdv,pdc->fvc", view, rhs,
                        precision=lax.Precision.DEFAULT).reshape(_NPAIR, _DP)
    xc32 = x_cat.astype(jnp.int32)
    offs = ((jnp.arange(_F, dtype=jnp.int32) // 2) * _V)[:, None]
    flat_idx = (xc32.T + offs).reshape(_ROWS)
    emb = _sc_gather(tab128, flat_idx)
    x = emb.reshape(_F, _B, _DP)
    return _tc_mlp(x, x_cont, g_cont, b_cont, W1, b1, g1, beta1, W2,
                   b2, g2, beta2, Wo, bo)
